# baseline (speedup 1.0000x reference)
import functools

import jax
import jax.numpy as jnp
from jax import lax
from jax.experimental import pallas as pl
from jax.experimental.pallas import tpu as pltpu
from jax.experimental.pallas import tpu_sc as plsc

_NUM_CORES = 2
_NUM_SUBCORES = 16
_NW = _NUM_CORES * _NUM_SUBCORES
_CHUNK = 64


_NBUF = 14


def _gather_body(cos_hbm, sin_hbm, idx_hbm, cos_out, sin_out,
                 idx_v, bufs, sem_g, sem_o, *, chunks_per_worker, workers_per_b):
    wid = lax.axis_index("s") * _NUM_CORES + lax.axis_index("c")
    b = wid // workers_per_b
    col = (wid % workers_per_b) * chunks_per_worker * _CHUNK
    pltpu.sync_copy(idx_hbm.at[b, pl.ds(col, chunks_per_worker * _CHUNK)], idx_v)

    work = [(cos_hbm, cos_out, j) for j in range(chunks_per_worker)]
    work += [(sin_hbm, sin_out, j) for j in range(chunks_per_worker)]
    nw = len(work)

    def buf(k):
        return bufs.at[k % _NBUF]

    def fire_gather(k):
        tab, _, j = work[k]
        pltpu.async_copy(tab.at[idx_v.at[pl.ds(j * _CHUNK, _CHUNK)]], buf(k),
                         sem_g)

    def drain_one(sem, k):
        pltpu.make_async_copy(cos_hbm.at[pl.ds(0, _CHUNK)], buf(k), sem).wait()

    for k in range(min(_NBUF, nw)):
        fire_gather(k)
    outs_fired = 0
    for k in range(nw):
        drain_one(sem_g, k)
        _, out, j = work[k]
        pltpu.async_copy(buf(k), out.at[b, pl.ds(col + j * _CHUNK, _CHUNK)],
                         sem_o)
        outs_fired += 1
        nk = k + _NBUF
        if nk < nw:
            drain_one(sem_o, nk)
            outs_fired -= 1
            fire_gather(nk)
    for k in range(outs_fired):
        drain_one(sem_o, k)


def kernel(x, position_ids, cos_cached, sin_cached):
    b, s = position_ids.shape
    dim = cos_cached.shape[-1]
    n = b * s
    assert n % (_NW * _CHUNK) == 0 and _NW % b == 0
    chunks_per_worker = n // (_NW * _CHUNK)
    workers_per_b = _NW // b

    cos_tab = cos_cached[0]
    sin_tab = sin_cached[0]

    mesh = plsc.VectorSubcoreMesh(core_axis_name="c", subcore_axis_name="s")
    out_t = jax.ShapeDtypeStruct((b, s, dim), jnp.float32)
    run = pl.kernel(
        functools.partial(_gather_body, chunks_per_worker=chunks_per_worker,
                          workers_per_b=workers_per_b),
        out_type=(out_t, out_t),
        mesh=mesh,
        scratch_types=[
            pltpu.VMEM((chunks_per_worker * _CHUNK,), jnp.int32),
            pltpu.VMEM((_NBUF, _CHUNK, dim), jnp.float32),
            pltpu.SemaphoreType.DMA,
            pltpu.SemaphoreType.DMA,
        ],
    )
    cos_o, sin_o = run(cos_tab, sin_tab, position_ids)
    return (cos_o.astype(x.dtype), sin_o.astype(x.dtype))


# speedup vs baseline: 1.0143x; 1.0143x over previous
import functools

import jax
import jax.numpy as jnp
from jax import lax
from jax.experimental import pallas as pl
from jax.experimental.pallas import tpu as pltpu
from jax.experimental.pallas import tpu_sc as plsc

_NUM_CORES = 2
_NUM_SUBCORES = 16
_NW = _NUM_CORES * _NUM_SUBCORES
_CHUNK = 128


_NBUF = 7


def _gather_body(cos_hbm, sin_hbm, idx_hbm, cos_out, sin_out,
                 idx_v, bufs, sem_g, sem_o, *, chunks_per_worker, workers_per_b):
    wid = lax.axis_index("s") * _NUM_CORES + lax.axis_index("c")
    b = wid // workers_per_b
    col = (wid % workers_per_b) * chunks_per_worker * _CHUNK
    pltpu.sync_copy(idx_hbm.at[b, pl.ds(col, chunks_per_worker * _CHUNK)], idx_v)

    work = [(cos_hbm, cos_out, j) for j in range(chunks_per_worker)]
    work += [(sin_hbm, sin_out, j) for j in range(chunks_per_worker)]
    nw = len(work)

    def buf(k):
        return bufs.at[k % _NBUF]

    def fire_gather(k):
        tab, _, j = work[k]
        pltpu.async_copy(
            tab.at[0].at[idx_v.at[pl.ds(j * _CHUNK, _CHUNK)]], buf(k), sem_g)

    def drain_one(sem, k):
        pltpu.make_async_copy(cos_hbm.at[0].at[pl.ds(0, _CHUNK)], buf(k),
                              sem).wait()

    for k in range(min(_NBUF, nw)):
        fire_gather(k)
    outs_fired = 0
    for k in range(nw):
        drain_one(sem_g, k)
        _, out, j = work[k]
        pltpu.async_copy(buf(k), out.at[b, pl.ds(col + j * _CHUNK, _CHUNK)],
                         sem_o)
        outs_fired += 1
        nk = k + _NBUF
        if nk < nw:
            drain_one(sem_o, nk)
            outs_fired -= 1
            fire_gather(nk)
    for k in range(outs_fired):
        drain_one(sem_o, k)


def kernel(x, position_ids, cos_cached, sin_cached):
    b, s = position_ids.shape
    dim = cos_cached.shape[-1]
    n = b * s
    assert n % (_NW * _CHUNK) == 0 and _NW % b == 0
    chunks_per_worker = n // (_NW * _CHUNK)
    workers_per_b = _NW // b

    mesh = plsc.VectorSubcoreMesh(core_axis_name="c", subcore_axis_name="s")
    out_t = jax.ShapeDtypeStruct((b, s, dim), jnp.float32)
    run = pl.kernel(
        functools.partial(_gather_body, chunks_per_worker=chunks_per_worker,
                          workers_per_b=workers_per_b),
        out_type=(out_t, out_t),
        mesh=mesh,
        scratch_types=[
            pltpu.VMEM((chunks_per_worker * _CHUNK,), jnp.int32),
            pltpu.VMEM((_NBUF, _CHUNK, dim), jnp.float32),
            pltpu.SemaphoreType.DMA,
            pltpu.SemaphoreType.DMA,
        ],
    )
    cos_o, sin_o = run(cos_cached, sin_cached, position_ids)
    return (cos_o.astype(x.dtype), sin_o.astype(x.dtype))
